# baseline (speedup 1.0000x reference)
import functools

import jax
import jax.numpy as jnp
from jax import lax
from jax.experimental import pallas as pl
from jax.experimental.pallas import tpu as pltpu
from jax.experimental.pallas import tpu_sc as plsc

_CHUNK = 128
_NC = 2
_NS = 16
_IBLK = 8
_NRB = 2


def _copy_plan(total):
    plan = []
    off = 0
    while off < total:
        sz = min(_CHUNK, total - off)
        plan.append((off, sz))
        off += sz
    return plan


@functools.lru_cache(maxsize=None)
def _make_agg(n, cap, d):
    nblk = cap // (_CHUNK * _IBLK * _NS * _NC)
    assert cap == nblk * _CHUNK * _IBLK * _NS * _NC
    assert nblk >= 2 and nblk % 2 == 0 and d % 16 == 0 and n % 8 == 0
    npairs = (nblk - 2) // 2
    base_rows = ((n + _NS - 1) // _NS + 7) // 8 * 8
    tail_rows = n - (_NS - 1) * base_rows
    assert 0 <= tail_rows <= base_rows and tail_rows % 8 == 0
    mesh = plsc.VectorSubcoreMesh(core_axis_name="c", subcore_axis_name="s")

    @functools.partial(
        pl.kernel,
        mesh=mesh,
        out_type=jax.ShapeDtypeStruct((_NC * n, d), jnp.float32),
        scratch_types=[
            pltpu.VMEM((2, _IBLK, _CHUNK), jnp.int32),
            pltpu.VMEM((2, _IBLK, _CHUNK), jnp.int32),
            pltpu.VMEM((_CHUNK, d), jnp.float32),
            pltpu.VMEM((_CHUNK, d), jnp.float32),
            pltpu.VMEM_SHARED((n + 8, d), jnp.float32),
            pltpu.SemaphoreType.DMA,
            pltpu.SemaphoreType.DMA,
            pltpu.SemaphoreType.DMA,
            pltpu.SemaphoreType.DMA,
            pltpu.SemaphoreType.DMA,
            pltpu.SemaphoreType.DMA,
        ],
    )
    def agg(h_hbm, ei_hbm, out_hbm, ibuf0, ibuf1,
            r0, r1, acc_sh, isem0, isem1, g0, g1, s0, s1):
        cid = lax.axis_index("c")
        sid = lax.axis_index("s")
        zbuf_v = r1
        rows = (r0, r1)
        gsem = (g0, g1)
        ssem = (s0, s1)
        ibuf = (ibuf0, ibuf1)
        isem = (isem0, isem1)

        zeros16 = jnp.zeros((16,), jnp.float32)

        def zrow(r, carry):
            for cc in range(d // 16):
                zbuf_v[r, pl.ds(cc * 16, 16)] = zeros16
            return carry

        lax.fori_loop(0, _CHUNK, zrow, None)
        row0 = sid * base_rows

        def zero_slice(nrows):
            for off, sz in _copy_plan(nrows):
                pltpu.sync_copy(zbuf_v.at[pl.ds(0, sz)],
                                acc_sh.at[pl.ds(row0 + off, sz)])

        @pl.when(sid < _NS - 1)
        def _():
            zero_slice(base_rows)

        @pl.when(sid == _NS - 1)
        def _():
            zero_slice(tail_rows + 8)

        plsc.subcore_barrier()

        tile_blk0 = (cid * _NS + sid) * nblk

        def idx_start(blk, s):
            pltpu.async_copy(ei_hbm.at[:, pl.ds(blk * _IBLK, _IBLK)],
                             ibuf[s], isem[s])

        def idx_wait(s):
            pltpu.make_async_copy(ei_hbm.at[:, pl.ds(0, _IBLK)],
                                  ibuf[s], isem[s]).wait()

        def gather_start(ib, k, b):
            pltpu.async_copy(h_hbm.at[ibuf[ib].at[0, k]], rows[b], gsem[b])

        def gather_wait(b):
            pltpu.make_async_copy(h_hbm.at[ibuf0.at[0, 0]],
                                  rows[b], gsem[b]).wait()

        def scat_start(ib, k, b):
            pltpu.async_copy(rows[b], acc_sh.at[ibuf[ib].at[1, k]],
                             ssem[b], add=True)

        def scat_wait(b):
            pltpu.make_async_copy(rows[b], acc_sh.at[ibuf0.at[1, 0]],
                                  ssem[b]).wait()

        idx_start(tile_blk0, 0)
        for blk in range(nblk):
            s = blk % 2
            idx_wait(s)
            if blk + 1 < nblk:
                idx_start(tile_blk0 + blk + 1, 1 - s)
            for k in range(_IBLK):
                gather_start(s, k, 0)
                gather_wait(0)
                scat_start(s, k, 0)
                scat_wait(0)
        plsc.subcore_barrier()

        out_row0 = cid * n + row0

        def write_slice(nrows):
            for off, sz in _copy_plan(nrows):
                pltpu.sync_copy(acc_sh.at[pl.ds(row0 + off, sz)],
                                zbuf_v.at[pl.ds(0, sz)])
                pltpu.sync_copy(zbuf_v.at[pl.ds(0, sz)],
                                out_hbm.at[pl.ds(out_row0 + off, sz)])

        @pl.when(sid < _NS - 1)
        def _():
            write_slice(base_rows)

        @pl.when(sid == _NS - 1)
        def _():
            write_slice(tail_rows)

    return agg


def _linear_body(x_ref, w_ref, b_ref, o_ref):
    o_ref[...] = (jnp.dot(x_ref[...], w_ref[...],
                          preferred_element_type=jnp.float32) + b_ref[...])


def _mlp_body(h_ref, a0_ref, a1_ref, w1_ref, b1_ref, w2_ref, b2_ref, o_ref):
    z = h_ref[...] + a0_ref[...] + a1_ref[...]
    t = jnp.maximum(jnp.dot(z, w1_ref[...],
                            preferred_element_type=jnp.float32) + b1_ref[...],
                    0.0)
    t = jnp.dot(t, w2_ref[...], preferred_element_type=jnp.float32) + b2_ref[...]
    o_ref[...] = jnp.maximum(t, 0.0)


def _row_block(n):
    for blk in (2000, 1000, 500, 250, 125):
        if n % blk == 0:
            return blk
    return n


def _linear(x, w, b):
    n, _ = x.shape
    d = w.shape[1]
    blk = _row_block(n)
    return pl.pallas_call(
        _linear_body,
        grid=(n // blk,),
        in_specs=[
            pl.BlockSpec((blk, x.shape[1]), lambda i: (i, 0)),
            pl.BlockSpec((x.shape[1], d), lambda i: (0, 0)),
            pl.BlockSpec((1, d), lambda i: (0, 0)),
        ],
        out_specs=pl.BlockSpec((blk, d), lambda i: (i, 0)),
        out_shape=jax.ShapeDtypeStruct((n, d), jnp.float32),
    )(x, w, b.reshape(1, d))


def _mlp(h, a0, a1, w1, b1, w2, b2):
    n, d = h.shape
    blk = _row_block(n)
    return pl.pallas_call(
        _mlp_body,
        grid=(n // blk,),
        in_specs=[
            pl.BlockSpec((blk, d), lambda i: (i, 0)),
            pl.BlockSpec((blk, d), lambda i: (i, 0)),
            pl.BlockSpec((blk, d), lambda i: (i, 0)),
            pl.BlockSpec((d, d), lambda i: (0, 0)),
            pl.BlockSpec((1, d), lambda i: (0, 0)),
            pl.BlockSpec((d, d), lambda i: (0, 0)),
            pl.BlockSpec((1, d), lambda i: (0, 0)),
        ],
        out_specs=pl.BlockSpec((blk, d), lambda i: (i, 0)),
        out_shape=jax.ShapeDtypeStruct((n, d), jnp.float32),
    )(h, a0, a1, w1, b1.reshape(1, d), w2, b2.reshape(1, d))


def kernel(x, edge_index, W_pre, b_pre, Ws1, bs1, Ws2, bs2):
    n = x.shape[0]
    d = W_pre.shape[1]
    e = edge_index.shape[1]
    layers = Ws1.shape[0]

    quantum = _CHUNK * _IBLK * _NS * _NC * 2
    cap = -(-e // quantum) * quantum
    pad = cap - e
    src = jnp.concatenate(
        [edge_index[0].astype(jnp.int32), jnp.zeros((pad,), jnp.int32)])
    dst = jnp.concatenate(
        [edge_index[1].astype(jnp.int32), jnp.full((pad,), n, jnp.int32)])
    ei3 = jnp.stack([src, dst]).reshape(2, cap // _CHUNK, _CHUNK)

    agg_fn = _make_agg(n, cap, d)
    h = _linear(x, W_pre, b_pre)
    for l in range(layers):
        parts = agg_fn(h, ei3)
        h = _mlp(h, parts[:n], parts[n:], Ws1[l], bs1[l], Ws2[l], bs2[l])
    return h


# speedup vs baseline: 2.4498x; 2.4498x over previous
import functools

import jax
import jax.numpy as jnp
from jax import lax
from jax.experimental import pallas as pl
from jax.experimental.pallas import tpu as pltpu
from jax.experimental.pallas import tpu_sc as plsc

_CHUNK = 128
_NC = 2
_NS = 16
_IBLK = 8
_NRB = 2


def _copy_plan(total):
    plan = []
    off = 0
    while off < total:
        sz = min(_CHUNK, total - off)
        plan.append((off, sz))
        off += sz
    return plan


@functools.lru_cache(maxsize=None)
def _make_agg(n, cap, d):
    nch = cap // _CHUNK
    assert cap == nch * _CHUNK and d % 16 == 0 and n % 8 == 0
    base_rows = ((n + _NS - 1) // _NS + 7) // 8 * 8
    tail_rows = n - (_NS - 1) * base_rows
    assert 0 <= tail_rows <= base_rows and tail_rows % 8 == 0
    mesh = plsc.VectorSubcoreMesh(core_axis_name="c", subcore_axis_name="s")

    @functools.partial(
        pl.kernel,
        mesh=mesh,
        out_type=jax.ShapeDtypeStruct((_NC * n, d), jnp.float32),
        scratch_types=[
            pltpu.VMEM((2, _IBLK, _CHUNK), jnp.int32),
            pltpu.VMEM((2, _IBLK, _CHUNK), jnp.int32),
            pltpu.VMEM((_CHUNK, d), jnp.float32),
            pltpu.VMEM((_CHUNK, d), jnp.float32),
            pltpu.VMEM_SHARED((n + 8, d), jnp.float32),
            pltpu.SemaphoreType.DMA,
            pltpu.SemaphoreType.DMA,
            pltpu.SemaphoreType.DMA,
            pltpu.SemaphoreType.DMA,
            pltpu.SemaphoreType.DMA,
            pltpu.SemaphoreType.DMA,
        ],
    )
    def agg(h_hbm, ei_hbm, out_hbm, ibuf0, ibuf1,
            r0, r1, acc_sh, isem0, isem1, g0, g1, s0, s1):
        cid = lax.axis_index("c")
        sid = lax.axis_index("s")
        zbuf_v = r1
        rows = (r0, r1)
        gsem = (g0, g1)
        ssem = (s0, s1)
        ibuf = (ibuf0, ibuf1)
        isem = (isem0, isem1)

        zeros16 = jnp.zeros((16,), jnp.float32)

        def zrow(r, carry):
            for cc in range(d // 16):
                zbuf_v[r, pl.ds(cc * 16, 16)] = zeros16
            return carry

        lax.fori_loop(0, _CHUNK, zrow, None)
        row0 = sid * base_rows

        def zero_slice(nrows):
            for off, sz in _copy_plan(nrows):
                pltpu.sync_copy(zbuf_v.at[pl.ds(0, sz)],
                                acc_sh.at[pl.ds(row0 + off, sz)])

        @pl.when(sid < _NS - 1)
        def _():
            zero_slice(base_rows)

        @pl.when(sid == _NS - 1)
        def _():
            zero_slice(tail_rows + 8)

        plsc.subcore_barrier()

        tl = cid * _NS + sid
        cnt = (nch - tl + _NS * _NC - 1) // (_NS * _NC)

        def chunk_body(i, carry):
            c = tl + i * (_NS * _NC)
            pltpu.sync_copy(ei_hbm.at[:, c], ibuf0.at[:, 0])
            pltpu.async_copy(h_hbm.at[ibuf0.at[0, 0]], r0, g0)
            pltpu.make_async_copy(h_hbm.at[ibuf0.at[0, 0]], r0, g0).wait()
            pltpu.async_copy(r0, acc_sh.at[ibuf0.at[1, 0]], s0, add=True)
            pltpu.make_async_copy(r0, acc_sh.at[ibuf0.at[1, 0]], s0).wait()
            return carry

        lax.fori_loop(0, cnt, chunk_body, None)
        plsc.subcore_barrier()

        out_row0 = cid * n + row0

        def write_slice(nrows):
            for off, sz in _copy_plan(nrows):
                pltpu.sync_copy(acc_sh.at[pl.ds(row0 + off, sz)],
                                zbuf_v.at[pl.ds(0, sz)])
                pltpu.sync_copy(zbuf_v.at[pl.ds(0, sz)],
                                out_hbm.at[pl.ds(out_row0 + off, sz)])

        @pl.when(sid < _NS - 1)
        def _():
            write_slice(base_rows)

        @pl.when(sid == _NS - 1)
        def _():
            write_slice(tail_rows)

    return agg


def _linear_body(x_ref, w_ref, b_ref, o_ref):
    o_ref[...] = (jnp.dot(x_ref[...], w_ref[...],
                          preferred_element_type=jnp.float32) + b_ref[...])


def _mlp_body(h_ref, a0_ref, a1_ref, w1_ref, b1_ref, w2_ref, b2_ref, o_ref):
    z = h_ref[...] + a0_ref[...] + a1_ref[...]
    t = jnp.maximum(jnp.dot(z, w1_ref[...],
                            preferred_element_type=jnp.float32) + b1_ref[...],
                    0.0)
    t = jnp.dot(t, w2_ref[...], preferred_element_type=jnp.float32) + b2_ref[...]
    o_ref[...] = jnp.maximum(t, 0.0)


def _row_block(n):
    for blk in (2000, 1000, 500, 250, 125):
        if n % blk == 0:
            return blk
    return n


def _linear(x, w, b):
    n, _ = x.shape
    d = w.shape[1]
    blk = _row_block(n)
    return pl.pallas_call(
        _linear_body,
        grid=(n // blk,),
        in_specs=[
            pl.BlockSpec((blk, x.shape[1]), lambda i: (i, 0)),
            pl.BlockSpec((x.shape[1], d), lambda i: (0, 0)),
            pl.BlockSpec((1, d), lambda i: (0, 0)),
        ],
        out_specs=pl.BlockSpec((blk, d), lambda i: (i, 0)),
        out_shape=jax.ShapeDtypeStruct((n, d), jnp.float32),
    )(x, w, b.reshape(1, d))


def _mlp(h, a0, a1, w1, b1, w2, b2):
    n, d = h.shape
    blk = _row_block(n)
    return pl.pallas_call(
        _mlp_body,
        grid=(n // blk,),
        in_specs=[
            pl.BlockSpec((blk, d), lambda i: (i, 0)),
            pl.BlockSpec((blk, d), lambda i: (i, 0)),
            pl.BlockSpec((blk, d), lambda i: (i, 0)),
            pl.BlockSpec((d, d), lambda i: (0, 0)),
            pl.BlockSpec((1, d), lambda i: (0, 0)),
            pl.BlockSpec((d, d), lambda i: (0, 0)),
            pl.BlockSpec((1, d), lambda i: (0, 0)),
        ],
        out_specs=pl.BlockSpec((blk, d), lambda i: (i, 0)),
        out_shape=jax.ShapeDtypeStruct((n, d), jnp.float32),
    )(h, a0, a1, w1, b1.reshape(1, d), w2, b2.reshape(1, d))


def kernel(x, edge_index, W_pre, b_pre, Ws1, bs1, Ws2, bs2):
    n = x.shape[0]
    d = W_pre.shape[1]
    e = edge_index.shape[1]
    layers = Ws1.shape[0]

    cap = -(-e // _CHUNK) * _CHUNK
    pad = cap - e
    src = jnp.concatenate(
        [edge_index[0].astype(jnp.int32), jnp.zeros((pad,), jnp.int32)])
    dst = jnp.concatenate(
        [edge_index[1].astype(jnp.int32), jnp.full((pad,), n, jnp.int32)])
    ei3 = jnp.stack([src, dst]).reshape(2, cap // _CHUNK, _CHUNK)

    agg_fn = _make_agg(n, cap, d)
    h = _linear(x, W_pre, b_pre)
    for l in range(layers):
        parts = agg_fn(h, ei3)
        h = _mlp(h, parts[:n], parts[n:], Ws1[l], bs1[l], Ws2[l], bs2[l])
    return h
